# baseline (speedup 1.0000x reference)
import functools

import jax
import jax.numpy as jnp
from jax.experimental import pallas as pl
from jax.experimental.pallas import tpu as pltpu

_E = 16
_D = 2048
_BD = 256
_NB = _D // _BD


def _moe_body(x_ref, wg_ref, bg_ref, wj_ref, bj_ref, z_ref, we_ref, be_ref,
              out_ref, xg_ref):
    e = pl.program_id(0)
    j = pl.program_id(1)

    @pl.when((e == 0) & (j == 0))
    def _router():
        xv = x_ref[...]
        logits = jnp.dot(xv, wg_ref[...],
                         preferred_element_type=jnp.float32) + bg_ref[...]
        pre = jnp.dot(xv, wj_ref[...],
                      preferred_element_type=jnp.float32) + bj_ref[...]
        scales = jax.nn.softplus(pre)
        t = logits + scales * z_ref[...]
        iota = jax.lax.broadcasted_iota(jnp.int32, (1, _E), 1)
        m1 = jnp.max(t)
        i1 = jnp.min(jnp.where(t == m1, iota, _E))
        masked = jnp.where(iota == i1, -jnp.inf, t)
        m2 = jnp.max(masked)
        i2 = jnp.min(jnp.where(masked == m2, iota, _E))
        sel = (iota == i1) | (iota == i2)
        sparse = jnp.where(sel, t, 0.0)
        g = jnp.exp(sparse - jnp.max(sparse))
        gate = g / jnp.sum(g)
        out_ref[...] = jnp.dot(gate, be_ref[...],
                               preferred_element_type=jnp.float32)
        xg_ref[...] = jax.lax.dot_general(
            gate, xv, dimension_numbers=(((0,), (0,)), ((), ())),
            preferred_element_type=jnp.float32)

    col = pl.multiple_of(j * _BD, _BD)
    xg_row = xg_ref[pl.ds(e, 1), pl.ds(col, _BD)]
    out_ref[...] += jnp.dot(xg_row, we_ref[0],
                            preferred_element_type=jnp.float32)


@jax.jit
def kernel(x, Wg, bg, Wj, bj, We, be, z):
    x2 = x.reshape(1, _D)
    bg2 = bg.reshape(1, _E)
    bj2 = bj.reshape(1, _E)
    z2 = z.reshape(1, _E)

    out = pl.pallas_call(
        _moe_body,
        grid=(_E, _NB),
        in_specs=[
            pl.BlockSpec((1, _D), lambda e, j: (0, 0)),
            pl.BlockSpec((_D, _E), lambda e, j: (0, 0)),
            pl.BlockSpec((1, _E), lambda e, j: (0, 0)),
            pl.BlockSpec((_D, _E), lambda e, j: (0, 0)),
            pl.BlockSpec((1, _E), lambda e, j: (0, 0)),
            pl.BlockSpec((1, _E), lambda e, j: (0, 0)),
            pl.BlockSpec((1, _BD, _D), lambda e, j: (e, j, 0)),
            pl.BlockSpec((_E, _D), lambda e, j: (0, 0)),
        ],
        out_specs=pl.BlockSpec((1, _D), lambda e, j: (0, 0)),
        out_shape=jax.ShapeDtypeStruct((1, _D), jnp.float32),
        scratch_shapes=[pltpu.VMEM((_E, _D), jnp.float32)],
    )(x2, Wg, bg2, Wj, bj2, z2, We, be)
    return out.reshape(_D)


# speedup vs baseline: 1.5530x; 1.5530x over previous
import functools

import jax
import jax.numpy as jnp
from jax.experimental import pallas as pl
from jax.experimental.pallas import tpu as pltpu

_E = 16
_D = 2048
_BD = 1024
_NB = _D // _BD


def _moe_body(x_ref, wg_ref, bg_ref, wj_ref, bj_ref, z_ref, we_ref, be_ref,
              out_ref, xg_ref):
    e = pl.program_id(0)
    j = pl.program_id(1)

    @pl.when((e == 0) & (j == 0))
    def _router():
        xv = x_ref[...]
        logits = jnp.dot(xv, wg_ref[...],
                         preferred_element_type=jnp.float32) + bg_ref[...]
        pre = jnp.dot(xv, wj_ref[...],
                      preferred_element_type=jnp.float32) + bj_ref[...]
        scales = jax.nn.softplus(pre)
        t = logits + scales * z_ref[...]
        iota = jax.lax.broadcasted_iota(jnp.int32, (1, _E), 1)
        m1 = jnp.max(t)
        i1 = jnp.min(jnp.where(t == m1, iota, _E))
        masked = jnp.where(iota == i1, -jnp.inf, t)
        m2 = jnp.max(masked)
        i2 = jnp.min(jnp.where(masked == m2, iota, _E))
        sel = (iota == i1) | (iota == i2)
        sparse = jnp.where(sel, t, 0.0)
        g = jnp.exp(sparse - jnp.max(sparse))
        gate = g / jnp.sum(g)
        out_ref[...] = jnp.dot(gate, be_ref[...],
                               preferred_element_type=jnp.float32)
        xg_ref[...] = jax.lax.dot_general(
            gate, xv, dimension_numbers=(((0,), (0,)), ((), ())),
            preferred_element_type=jnp.float32)

    col = pl.multiple_of(j * _BD, _BD)
    xg_row = xg_ref[pl.ds(e, 1), pl.ds(col, _BD)]
    out_ref[...] += jnp.dot(xg_row, we_ref[0],
                            preferred_element_type=jnp.float32)


@jax.jit
def kernel(x, Wg, bg, Wj, bj, We, be, z):
    x2 = x.reshape(1, _D)
    bg2 = bg.reshape(1, _E)
    bj2 = bj.reshape(1, _E)
    z2 = z.reshape(1, _E)

    out = pl.pallas_call(
        _moe_body,
        grid=(_E, _NB),
        in_specs=[
            pl.BlockSpec((1, _D), lambda e, j: (0, 0)),
            pl.BlockSpec((_D, _E), lambda e, j: (0, 0)),
            pl.BlockSpec((1, _E), lambda e, j: (0, 0)),
            pl.BlockSpec((_D, _E), lambda e, j: (0, 0)),
            pl.BlockSpec((1, _E), lambda e, j: (0, 0)),
            pl.BlockSpec((1, _E), lambda e, j: (0, 0)),
            pl.BlockSpec((1, _BD, _D), lambda e, j: (e, j, 0)),
            pl.BlockSpec((_E, _D), lambda e, j: (0, 0)),
        ],
        out_specs=pl.BlockSpec((1, _D), lambda e, j: (0, 0)),
        out_shape=jax.ShapeDtypeStruct((1, _D), jnp.float32),
        scratch_shapes=[pltpu.VMEM((_E, _D), jnp.float32)],
    )(x2, Wg, bg2, Wj, bj2, z2, We, be)
    return out.reshape(_D)
